# baseline (speedup 1.0000x reference)
import functools

import jax
import jax.numpy as jnp
from jax import lax
from jax.experimental import pallas as pl
from jax.experimental.pallas import tpu as pltpu
from jax.experimental.pallas import tpu_sc as plsc

_BLOCK_TOKENS = 4096
_NUM_SELECTS = 8
_NE = 64
_TILE_TOKENS = 16


def _matmul_kernel(x_ref, w_ref, out_ref):
    out_ref[...] = jax.lax.dot_general(
        x_ref[...], w_ref[...], (((1,), (1,)), ((), ())),
        preferred_element_type=jnp.float32,
    )


def _max16(v):
    dnums = lax.GatherDimensionNumbers(
        offset_dims=(), collapsed_slice_dims=(0,), start_index_map=(0,)
    )
    for s in (1, 2, 4, 8):
        idx = lax.iota(jnp.int32, 16) ^ s
        shuf = lax.gather(
            v,
            idx[:, None],
            dimension_numbers=dnums,
            slice_sizes=(1,),
            mode=lax.GatherScatterMode.PROMISE_IN_BOUNDS,
        )
        v = jnp.maximum(v, shuf)
    return v


def _topk_mask_sc(num_tokens):
    info = plsc.get_sparse_core_info()
    nw = info.num_cores * info.num_subcores
    rows_per_w = num_tokens // nw
    tiles_per_w = rows_per_w // _TILE_TOKENS
    tile_elems = _TILE_TOKENS * _NE
    mesh = plsc.VectorSubcoreMesh(core_axis_name="c", subcore_axis_name="s")

    @functools.partial(
        pl.kernel,
        mesh=mesh,
        out_type=jax.ShapeDtypeStruct((num_tokens * _NE,), jnp.float32),
        scratch_types=[
            pltpu.VMEM((tile_elems,), jnp.float32),
            pltpu.VMEM((tile_elems,), jnp.float32),
        ],
    )
    def topk(logits_hbm, out_hbm, tile, otile):
        wid = lax.axis_index("s") * info.num_cores + lax.axis_index("c")
        base = wid * rows_per_w * _NE

        def tile_body(j, carry):
            off = base + j * tile_elems
            pltpu.sync_copy(logits_hbm.at[pl.ds(off, tile_elems)], tile)

            def tok_body(tok, c):
                tb = tok * _NE
                v0 = tile[pl.ds(tb, 16)]
                v1 = tile[pl.ds(tb + 16, 16)]
                v2 = tile[pl.ds(tb + 32, 16)]
                v3 = tile[pl.ds(tb + 48, 16)]
                a, b, cc, d = v0, v1, v2, v3
                neg = jnp.float32(-jnp.inf)
                for _ in range(_NUM_SELECTS - 1):
                    m = _max16(
                        jnp.maximum(jnp.maximum(a, b), jnp.maximum(cc, d))
                    )
                    a = jnp.where(a == m, neg, a)
                    b = jnp.where(b == m, neg, b)
                    cc = jnp.where(cc == m, neg, cc)
                    d = jnp.where(d == m, neg, d)
                t = _max16(
                    jnp.maximum(jnp.maximum(a, b), jnp.maximum(cc, d))
                )
                one = jnp.float32(1.0)
                zero = jnp.float32(0.0)
                otile[pl.ds(tb, 16)] = jnp.where(v0 >= t, one, zero)
                otile[pl.ds(tb + 16, 16)] = jnp.where(v1 >= t, one, zero)
                otile[pl.ds(tb + 32, 16)] = jnp.where(v2 >= t, one, zero)
                otile[pl.ds(tb + 48, 16)] = jnp.where(v3 >= t, one, zero)
                return c

            lax.fori_loop(0, _TILE_TOKENS, tok_body, 0)
            pltpu.sync_copy(otile, out_hbm.at[pl.ds(off, tile_elems)])
            return carry

        lax.fori_loop(0, tiles_per_w, tile_body, 0)

    return topk


def kernel(routing_inputs, w_gate):
    num_tokens, hidden = routing_inputs.shape
    num_experts = w_gate.shape[0]
    bt = min(_BLOCK_TOKENS, num_tokens)
    grid = (num_tokens // bt,)
    logits = pl.pallas_call(
        _matmul_kernel,
        grid=grid,
        in_specs=[
            pl.BlockSpec((bt, hidden), lambda i: (i, 0)),
            pl.BlockSpec((num_experts, hidden), lambda i: (0, 0)),
        ],
        out_specs=pl.BlockSpec((bt, num_experts), lambda i: (i, 0)),
        out_shape=jax.ShapeDtypeStruct((num_tokens, num_experts), jnp.float32),
    )(routing_inputs, w_gate)
    mask_flat = _topk_mask_sc(num_tokens)(logits.reshape(-1))
    return mask_flat.reshape(num_tokens, num_experts)


# speedup vs baseline: 2.9213x; 2.9213x over previous
import jax
import jax.numpy as jnp
from jax.experimental import pallas as pl

_NUM_SELECTS = 8
_BLOCK_TOKENS = 2048


def _gate_kernel(x_ref, w_ref, out_ref):
    x = x_ref[...]
    w = w_ref[...]
    dims = (((1,), (1,)), ((), ()))
    logits_t = jax.lax.dot_general(
        w, x, dims, preferred_element_type=jnp.float32
    )
    neg_inf = jnp.float32(-jnp.inf)
    work = logits_t
    for _ in range(_NUM_SELECTS - 1):
        mx = jnp.max(work, axis=0, keepdims=True)
        work = jnp.where(work == mx, neg_inf, work)
    t = jnp.max(work, axis=0, keepdims=True)
    mask_t = jnp.where(logits_t >= t, jnp.float32(1.0), jnp.float32(0.0))
    out_ref[...] = jnp.transpose(mask_t)


def kernel(routing_inputs, w_gate):
    num_tokens, hidden = routing_inputs.shape
    num_experts = w_gate.shape[0]
    bt = min(_BLOCK_TOKENS, num_tokens)
    grid = (num_tokens // bt,)
    return pl.pallas_call(
        _gate_kernel,
        grid=grid,
        in_specs=[
            pl.BlockSpec((bt, hidden), lambda i: (i, 0)),
            pl.BlockSpec((num_experts, hidden), lambda i: (0, 0)),
        ],
        out_specs=pl.BlockSpec((bt, num_experts), lambda i: (i, 0)),
        out_shape=jax.ShapeDtypeStruct((num_tokens, num_experts), jnp.float32),
    )(routing_inputs, w_gate)


# speedup vs baseline: 2.9652x; 1.0150x over previous
import jax
import jax.numpy as jnp
from jax.experimental import pallas as pl

_NUM_SELECTS = 8
_BLOCK_TOKENS = 4096


def _gate_kernel(x_ref, w_ref, out_ref):
    x = x_ref[...]
    w = w_ref[...]
    dims = (((1,), (1,)), ((), ()))
    logits_t = jax.lax.dot_general(
        w, x, dims, preferred_element_type=jnp.float32
    )
    neg_inf = jnp.float32(-jnp.inf)
    work = logits_t
    for _ in range(_NUM_SELECTS - 1):
        mx = jnp.max(work, axis=0, keepdims=True)
        work = jnp.where(work == mx, neg_inf, work)
    t = jnp.max(work, axis=0, keepdims=True)
    mask_t = jnp.where(logits_t >= t, jnp.float32(1.0), jnp.float32(0.0))
    out_ref[...] = jnp.transpose(mask_t)


def kernel(routing_inputs, w_gate):
    num_tokens, hidden = routing_inputs.shape
    num_experts = w_gate.shape[0]
    bt = min(_BLOCK_TOKENS, num_tokens)
    grid = (num_tokens // bt,)
    return pl.pallas_call(
        _gate_kernel,
        grid=grid,
        in_specs=[
            pl.BlockSpec((bt, hidden), lambda i: (i, 0)),
            pl.BlockSpec((num_experts, hidden), lambda i: (0, 0)),
        ],
        out_specs=pl.BlockSpec((bt, num_experts), lambda i: (i, 0)),
        out_shape=jax.ShapeDtypeStruct((num_tokens, num_experts), jnp.float32),
    )(routing_inputs, w_gate)
